# baseline (speedup 1.0000x reference)
import functools

import jax
import jax.numpy as jnp
from jax.experimental import pallas as pl
from jax.experimental.pallas import tpu as pltpu

N_FEAT = 64
WD = 128
B = 32
H = 64
CS = 8
S = 4


def _conv_kernel(xp_ref, w2_ref, b2_ref, w3_ref, b3_ref, o2_ref, o3_ref):
    for b in range(CS):
        xp = xp_ref[b]
        acc2 = jnp.broadcast_to(b2_ref[0], (WD, N_FEAT))
        for k in range(5):
            acc2 = acc2 + jnp.dot(xp[k + 1:k + 1 + WD, :], w2_ref[k],
                                  preferred_element_type=jnp.float32)
        o2_ref[b] = jnp.maximum(acc2, 0.0)
        acc3 = jnp.broadcast_to(b3_ref[0], (WD, N_FEAT))
        for k in range(7):
            acc3 = acc3 + jnp.dot(xp[k:k + WD, :], w3_ref[k],
                                  preferred_element_type=jnp.float32)
        o3_ref[b] = jnp.maximum(acc3, 0.0)


def _one_block(d, gwt, gas, gad, gb, cwt, cb, tri):
    xn = d.T
    h = jnp.dot(xn, gwt, preferred_element_type=jnp.float32)
    ht = h.T
    a_src = jnp.dot(gas, ht, preferred_element_type=jnp.float32)
    a_dst = jnp.dot(h, gad, preferred_element_type=jnp.float32)
    logits = a_dst + a_src
    logits = jnp.where(logits > 0, logits, 0.2 * logits)
    m = jnp.max(logits, axis=1, keepdims=True)
    e = jnp.exp(logits - m)
    att = e / jnp.sum(e, axis=1, keepdims=True)
    f = jnp.maximum(jnp.dot(att, h, preferred_element_type=jnp.float32)
                    + gb, 0.0)
    tin = f.T
    hh = jnp.dot(tin, cwt, preferred_element_type=jnp.float32)
    g = jnp.dot(tri, hh, preferred_element_type=jnp.float32)
    v = jnp.maximum(g + cb, 0.0)
    v3 = v.reshape(64, 2, 64)
    e0 = v3[:, 0, :].reshape(64, 64)
    e1 = v3[:, 1, :].reshape(64, 64)
    return jnp.concatenate([e0.T, e1.T], axis=0)


def _block_kernel(d_ref, gwt0, gas0, gad0, gb0, cwt0, cb0,
                  gwt1, gas1, gad1, gb1, cwt1, cb1, tri_ref, o_ref):
    tri = tri_ref[...]
    for b in range(S):
        d = d_ref[b]
        d = d + _one_block(d, gwt0[0], gas0[0], gad0[0], gb0[0],
                           cwt0[0], cb0[0], tri)
        d = d + _one_block(d, gwt1[0], gas1[0], gad1[0], gb1[0],
                           cwt1[0], cb1[0], tri)
        o_ref[b] = d


def _sigmoid(v):
    return jax.nn.sigmoid(v)


def _lstm_kernel(x_ref, wih_f, whh_f, bias_f, wih_b, bias_b,
                 rwhh_f, rbias_f, rwsum_f, rwhh_b, rbias_b, rwsum_b,
                 fca_ref, fcb_ref, fcbias_ref,
                 out_ref, xg_ref, hsf_ref, hsb_ref):
    for i in range(16):
        xg_ref[i * 256:(i + 1) * 256, :] = jnp.dot(
            x_ref[i * 256:(i + 1) * 256, :], wih_f[...],
            preferred_element_type=jnp.float32)

    zero = jnp.zeros((B, H), jnp.float32)

    def lstm_update(g, c):
        i = _sigmoid(g[:, 0:H])
        f = _sigmoid(g[:, H:2 * H])
        gg = jnp.tanh(g[:, 2 * H:3 * H])
        o = _sigmoid(g[:, 3 * H:4 * H])
        c2 = f * c + i * gg
        return o * jnp.tanh(c2), c2

    def enc_step(t, carry):
        h, c = carry
        g = (xg_ref[pl.ds(t * B, B), :]
             + jnp.dot(h, whh_f[...], preferred_element_type=jnp.float32)
             + bias_f[...])
        return lstm_update(g, c)

    h_f, _ = jax.lax.fori_loop(0, WD, enc_step, (zero, zero))

    gb = jnp.dot(x_ref[(WD - 1) * B:WD * B, :], wih_b[...],
                 preferred_element_type=jnp.float32) + bias_b[...]
    h_b, _ = lstm_update(gb, jnp.zeros((B, H), jnp.float32))

    ue = jnp.concatenate([h_f, h_b], axis=1)

    lane = jax.lax.broadcasted_iota(jnp.int32, (B, 2 * H), 1)

    def col(t):
        return jnp.sum(jnp.where(lane == t, ue, 0.0), axis=1, keepdims=True)

    def dec_step(k, carry):
        hf, cf, hb, cb = carry
        gf = (col(k) * rwsum_f[...]
              + jnp.dot(hf, rwhh_f[...], preferred_element_type=jnp.float32)
              + rbias_f[...])
        hf, cf = lstm_update(gf, cf)
        gbk = (col(WD - 1 - k) * rwsum_b[...]
               + jnp.dot(hb, rwhh_b[...], preferred_element_type=jnp.float32)
               + rbias_b[...])
        hb, cb = lstm_update(gbk, cb)
        hsf_ref[pl.ds(k * B, B), :] = hf
        hsb_ref[pl.ds((WD - 1 - k) * B, B), :] = hb
        return hf, cf, hb, cb

    jax.lax.fori_loop(0, WD, dec_step, (zero, zero, zero, zero))

    for i in range(8):
        sl = pl.ds(i * 512, 512)
        out_ref[sl, :] = (
            jnp.dot(hsf_ref[sl, :], fca_ref[...],
                    preferred_element_type=jnp.float32)
            + jnp.dot(hsb_ref[sl, :], fcb_ref[...],
                      preferred_element_type=jnp.float32)
            + fcbias_ref[...])


@functools.partial(jax.jit, static_argnames=())
def kernel(x, conv2_W, conv2_b, conv3_W, conv3_b, gat_W, gat_as, gat_ad,
           gat_b, gcn_W, gcn_b, lstm_Wih, lstm_Whh, lstm_bih, lstm_bhh,
           rec_Wih, rec_Whh, rec_bih, rec_bhh, fc_W, fc_b):
    f32 = jnp.float32

    xp = jnp.pad(x, ((0, 0), (3, 3), (0, 0)))
    w2t = jnp.transpose(conv2_W, (2, 1, 0))
    w3t = jnp.transpose(conv3_W, (2, 1, 0))
    b2 = conv2_b.reshape(1, 1, N_FEAT)
    b3 = conv3_b.reshape(1, 1, N_FEAT)
    x2, x3 = pl.pallas_call(
        _conv_kernel,
        grid=(B // CS,),
        in_specs=[
            pl.BlockSpec((CS, WD + 6, N_FEAT), lambda i: (i, 0, 0)),
            pl.BlockSpec((5, N_FEAT, N_FEAT), lambda i: (0, 0, 0)),
            pl.BlockSpec((1, 1, N_FEAT), lambda i: (0, 0, 0)),
            pl.BlockSpec((7, N_FEAT, N_FEAT), lambda i: (0, 0, 0)),
            pl.BlockSpec((1, 1, N_FEAT), lambda i: (0, 0, 0)),
        ],
        out_specs=[
            pl.BlockSpec((CS, WD, N_FEAT), lambda i: (i, 0, 0)),
            pl.BlockSpec((CS, WD, N_FEAT), lambda i: (i, 0, 0)),
        ],
        out_shape=[
            jax.ShapeDtypeStruct((B, WD, N_FEAT), f32),
            jax.ShapeDtypeStruct((B, WD, N_FEAT), f32),
        ],
    )(xp, w2t, b2, w3t, b3)

    idx = jnp.arange(WD, dtype=f32)
    dinv = (idx + 1.0) ** -0.5
    tri = jnp.tril(jnp.ones((WD, WD), f32)) * (dinv[:, None] * dinv[None, :])

    data = jnp.stack([x, x2, x3]).reshape(3 * B, WD, N_FEAT)
    gwt = jnp.transpose(gat_W, (0, 2, 1))
    cwt = jnp.transpose(gcn_W, (0, 2, 1))

    nprog = 3 * B // S
    per_branch = B // S

    def wspec(shape):
        return pl.BlockSpec((1,) + shape, lambda i: (i // per_branch, 0, 0))

    def layer_args(l):
        return (
            gwt[l::2],
            gat_as[l::2].reshape(3, 1, WD),
            gat_ad[l::2].reshape(3, WD, 1),
            gat_b[l::2].reshape(3, 1, WD),
            cwt[l::2],
            gcn_b[l::2].reshape(3, 1, N_FEAT),
        )

    def layer_specs():
        return [
            wspec((WD, WD)),
            wspec((1, WD)),
            wspec((WD, 1)),
            wspec((1, WD)),
            wspec((N_FEAT, N_FEAT)),
            wspec((1, N_FEAT)),
        ]

    data = pl.pallas_call(
        _block_kernel,
        grid=(nprog,),
        in_specs=(
            [pl.BlockSpec((S, WD, N_FEAT), lambda i: (i, 0, 0))]
            + layer_specs() + layer_specs()
            + [pl.BlockSpec((WD, WD), lambda i: (0, 0))]
        ),
        out_specs=pl.BlockSpec((S, WD, N_FEAT), lambda i: (i, 0, 0)),
        out_shape=jax.ShapeDtypeStruct((3 * B, WD, N_FEAT), f32),
    )(data, *layer_args(0), *layer_args(1), tri)

    xs = data.reshape(3, B, WD, N_FEAT).transpose(2, 1, 0, 3)
    xs = xs.reshape(WD * B, 3 * N_FEAT)

    wih_f = lstm_Wih[0].T
    whh_f = lstm_Whh[0].T
    bias_f = (lstm_bih[0] + lstm_bhh[0]).reshape(1, 4 * H)
    wih_b = lstm_Wih[1].T
    bias_b = (lstm_bih[1] + lstm_bhh[1]).reshape(1, 4 * H)

    rwhh_f = rec_Whh[0].T
    rbias_f = (rec_bih[0] + rec_bhh[0]).reshape(1, 4 * H)
    rwsum_f = jnp.sum(rec_Wih[0], axis=1).reshape(1, 4 * H)
    rwhh_b = rec_Whh[1].T
    rbias_b = (rec_bih[1] + rec_bhh[1]).reshape(1, 4 * H)
    rwsum_b = jnp.sum(rec_Wih[1], axis=1).reshape(1, 4 * H)

    fca = fc_W[:, :H].T
    fcb = fc_W[:, H:].T
    fcbias = fc_b.reshape(1, N_FEAT)

    out = pl.pallas_call(
        _lstm_kernel,
        out_shape=jax.ShapeDtypeStruct((WD * B, N_FEAT), f32),
        scratch_shapes=[
            pltpu.VMEM((WD * B, 4 * H), f32),
            pltpu.VMEM((WD * B, H), f32),
            pltpu.VMEM((WD * B, H), f32),
        ],
    )(xs, wih_f, whh_f, bias_f, wih_b, bias_b,
      rwhh_f, rbias_f, rwsum_f, rwhh_b, rbias_b, rwsum_b,
      fca, fcb, fcbias)

    return out.reshape(WD, B, N_FEAT).transpose(1, 0, 2)


# speedup vs baseline: 1.0006x; 1.0006x over previous
import functools

import jax
import jax.numpy as jnp
from jax.experimental import pallas as pl
from jax.experimental.pallas import tpu as pltpu

N_FEAT = 64
WD = 128
B = 32
H = 64
CS = 8
S = 4


def _conv_kernel(xp_ref, w2_ref, b2_ref, w3_ref, b3_ref, o2_ref, o3_ref):
    for b in range(CS):
        xp = xp_ref[b]
        acc2 = jnp.broadcast_to(b2_ref[0], (WD, N_FEAT))
        for k in range(5):
            acc2 = acc2 + jnp.dot(xp[k + 1:k + 1 + WD, :], w2_ref[k],
                                  preferred_element_type=jnp.float32)
        o2_ref[b] = jnp.maximum(acc2, 0.0)
        acc3 = jnp.broadcast_to(b3_ref[0], (WD, N_FEAT))
        for k in range(7):
            acc3 = acc3 + jnp.dot(xp[k:k + WD, :], w3_ref[k],
                                  preferred_element_type=jnp.float32)
        o3_ref[b] = jnp.maximum(acc3, 0.0)


def _one_block(d, gwt, gas, gad, gb, cwt, cb, tri):
    xn = d.T
    h = jnp.dot(xn, gwt, preferred_element_type=jnp.float32)
    ht = h.T
    a_src = jnp.dot(gas, ht, preferred_element_type=jnp.float32)
    a_dst = jnp.dot(h, gad, preferred_element_type=jnp.float32)
    logits = a_dst + a_src
    logits = jnp.where(logits > 0, logits, 0.2 * logits)
    m = jnp.max(logits, axis=1, keepdims=True)
    e = jnp.exp(logits - m)
    att = e / jnp.sum(e, axis=1, keepdims=True)
    f = jnp.maximum(jnp.dot(att, h, preferred_element_type=jnp.float32)
                    + gb, 0.0)
    tin = f.T
    hh = jnp.dot(tin, cwt, preferred_element_type=jnp.float32)
    g = jnp.dot(tri, hh, preferred_element_type=jnp.float32)
    v = jnp.maximum(g + cb, 0.0)
    v3 = v.reshape(64, 2, 64)
    e0 = v3[:, 0, :].reshape(64, 64)
    e1 = v3[:, 1, :].reshape(64, 64)
    return jnp.concatenate([e0.T, e1.T], axis=0)


def _block_kernel(d_ref, gwt0, gas0, gad0, gb0, cwt0, cb0,
                  gwt1, gas1, gad1, gb1, cwt1, cb1, tri_ref, o_ref):
    tri = tri_ref[...]
    for b in range(S):
        d = d_ref[b]
        d = d + _one_block(d, gwt0[0], gas0[0], gad0[0], gb0[0],
                           cwt0[0], cb0[0], tri)
        d = d + _one_block(d, gwt1[0], gas1[0], gad1[0], gb1[0],
                           cwt1[0], cb1[0], tri)
        o_ref[b] = d


def _sigmoid(v):
    return jax.nn.sigmoid(v)


def _lstm_kernel(x_ref, wih_f, whh_f, bias_f, wih_b, bias_b,
                 rwhh_f, rbias_f, rwsum_f, rwhh_b, rbias_b, rwsum_b,
                 fca_ref, fcb_ref, fcbias_ref,
                 out_ref, xg_ref, hsf_ref, hsb_ref):
    for i in range(16):
        xg_ref[i * 256:(i + 1) * 256, :] = jnp.dot(
            x_ref[i * 256:(i + 1) * 256, :], wih_f[...],
            preferred_element_type=jnp.float32)

    zero = jnp.zeros((B, H), jnp.float32)

    def lstm_update(g, c):
        i = _sigmoid(g[:, 0:H])
        f = _sigmoid(g[:, H:2 * H])
        gg = jnp.tanh(g[:, 2 * H:3 * H])
        o = _sigmoid(g[:, 3 * H:4 * H])
        c2 = f * c + i * gg
        return o * jnp.tanh(c2), c2

    def enc_step(t, carry):
        h, c = carry
        g = (xg_ref[pl.ds(t * B, B), :]
             + jnp.dot(h, whh_f[...], preferred_element_type=jnp.float32)
             + bias_f[...])
        return lstm_update(g, c)

    h_f, _ = jax.lax.fori_loop(0, WD, enc_step, (zero, zero))

    gb = jnp.dot(x_ref[(WD - 1) * B:WD * B, :], wih_b[...],
                 preferred_element_type=jnp.float32) + bias_b[...]
    h_b, _ = lstm_update(gb, jnp.zeros((B, H), jnp.float32))

    ue = jnp.concatenate([h_f, h_b], axis=1)

    lane = jax.lax.broadcasted_iota(jnp.int32, (B, 2 * H), 1)

    def col(t):
        return jnp.sum(jnp.where(lane == t, ue, 0.0), axis=1, keepdims=True)

    def dec_step(k, carry):
        hf, cf, hb, cb = carry
        gf = (col(k) * rwsum_f[...]
              + jnp.dot(hf, rwhh_f[...], preferred_element_type=jnp.float32)
              + rbias_f[...])
        hf, cf = lstm_update(gf, cf)
        gbk = (col(WD - 1 - k) * rwsum_b[...]
               + jnp.dot(hb, rwhh_b[...], preferred_element_type=jnp.float32)
               + rbias_b[...])
        hb, cb = lstm_update(gbk, cb)
        hsf_ref[pl.ds(k * B, B), :] = hf
        hsb_ref[pl.ds((WD - 1 - k) * B, B), :] = hb
        return hf, cf, hb, cb

    jax.lax.fori_loop(0, WD, dec_step, (zero, zero, zero, zero))

    for i in range(8):
        sl = pl.ds(i * 512, 512)
        out_ref[sl, :] = (
            jnp.dot(hsf_ref[sl, :], fca_ref[...],
                    preferred_element_type=jnp.float32)
            + jnp.dot(hsb_ref[sl, :], fcb_ref[...],
                      preferred_element_type=jnp.float32)
            + fcbias_ref[...])


@functools.partial(jax.jit, static_argnames=())
def kernel(x, conv2_W, conv2_b, conv3_W, conv3_b, gat_W, gat_as, gat_ad,
           gat_b, gcn_W, gcn_b, lstm_Wih, lstm_Whh, lstm_bih, lstm_bhh,
           rec_Wih, rec_Whh, rec_bih, rec_bhh, fc_W, fc_b):
    f32 = jnp.float32

    xp = jnp.pad(x, ((0, 0), (3, 3), (0, 0)))
    w2t = jnp.transpose(conv2_W, (2, 1, 0))
    w3t = jnp.transpose(conv3_W, (2, 1, 0))
    b2 = conv2_b.reshape(1, 1, N_FEAT)
    b3 = conv3_b.reshape(1, 1, N_FEAT)
    x2, x3 = pl.pallas_call(
        _conv_kernel,
        grid=(B // CS,),
        in_specs=[
            pl.BlockSpec((CS, WD + 6, N_FEAT), lambda i: (i, 0, 0)),
            pl.BlockSpec((5, N_FEAT, N_FEAT), lambda i: (0, 0, 0)),
            pl.BlockSpec((1, 1, N_FEAT), lambda i: (0, 0, 0)),
            pl.BlockSpec((7, N_FEAT, N_FEAT), lambda i: (0, 0, 0)),
            pl.BlockSpec((1, 1, N_FEAT), lambda i: (0, 0, 0)),
        ],
        out_specs=[
            pl.BlockSpec((CS, WD, N_FEAT), lambda i: (i, 0, 0)),
            pl.BlockSpec((CS, WD, N_FEAT), lambda i: (i, 0, 0)),
        ],
        out_shape=[
            jax.ShapeDtypeStruct((B, WD, N_FEAT), f32),
            jax.ShapeDtypeStruct((B, WD, N_FEAT), f32),
        ],
        compiler_params=pltpu.CompilerParams(
            dimension_semantics=("parallel",)),
    )(xp, w2t, b2, w3t, b3)

    idx = jnp.arange(WD, dtype=f32)
    dinv = (idx + 1.0) ** -0.5
    tri = jnp.tril(jnp.ones((WD, WD), f32)) * (dinv[:, None] * dinv[None, :])

    data = jnp.stack([x, x2, x3]).reshape(3 * B, WD, N_FEAT)
    gwt = jnp.transpose(gat_W, (0, 2, 1))
    cwt = jnp.transpose(gcn_W, (0, 2, 1))

    nprog = 3 * B // S
    per_branch = B // S

    def wspec(shape):
        return pl.BlockSpec((1,) + shape, lambda i: (i // per_branch, 0, 0))

    def layer_args(l):
        return (
            gwt[l::2],
            gat_as[l::2].reshape(3, 1, WD),
            gat_ad[l::2].reshape(3, WD, 1),
            gat_b[l::2].reshape(3, 1, WD),
            cwt[l::2],
            gcn_b[l::2].reshape(3, 1, N_FEAT),
        )

    def layer_specs():
        return [
            wspec((WD, WD)),
            wspec((1, WD)),
            wspec((WD, 1)),
            wspec((1, WD)),
            wspec((N_FEAT, N_FEAT)),
            wspec((1, N_FEAT)),
        ]

    data = pl.pallas_call(
        _block_kernel,
        grid=(nprog,),
        in_specs=(
            [pl.BlockSpec((S, WD, N_FEAT), lambda i: (i, 0, 0))]
            + layer_specs() + layer_specs()
            + [pl.BlockSpec((WD, WD), lambda i: (0, 0))]
        ),
        out_specs=pl.BlockSpec((S, WD, N_FEAT), lambda i: (i, 0, 0)),
        out_shape=jax.ShapeDtypeStruct((3 * B, WD, N_FEAT), f32),
        compiler_params=pltpu.CompilerParams(
            dimension_semantics=("parallel",)),
    )(data, *layer_args(0), *layer_args(1), tri)

    xs = data.reshape(3, B, WD, N_FEAT).transpose(2, 1, 0, 3)
    xs = xs.reshape(WD * B, 3 * N_FEAT)

    wih_f = lstm_Wih[0].T
    whh_f = lstm_Whh[0].T
    bias_f = (lstm_bih[0] + lstm_bhh[0]).reshape(1, 4 * H)
    wih_b = lstm_Wih[1].T
    bias_b = (lstm_bih[1] + lstm_bhh[1]).reshape(1, 4 * H)

    rwhh_f = rec_Whh[0].T
    rbias_f = (rec_bih[0] + rec_bhh[0]).reshape(1, 4 * H)
    rwsum_f = jnp.sum(rec_Wih[0], axis=1).reshape(1, 4 * H)
    rwhh_b = rec_Whh[1].T
    rbias_b = (rec_bih[1] + rec_bhh[1]).reshape(1, 4 * H)
    rwsum_b = jnp.sum(rec_Wih[1], axis=1).reshape(1, 4 * H)

    fca = fc_W[:, :H].T
    fcb = fc_W[:, H:].T
    fcbias = fc_b.reshape(1, N_FEAT)

    out = pl.pallas_call(
        _lstm_kernel,
        out_shape=jax.ShapeDtypeStruct((WD * B, N_FEAT), f32),
        scratch_shapes=[
            pltpu.VMEM((WD * B, 4 * H), f32),
            pltpu.VMEM((WD * B, H), f32),
            pltpu.VMEM((WD * B, H), f32),
        ],
    )(xs, wih_f, whh_f, bias_f, wih_b, bias_b,
      rwhh_f, rbias_f, rwsum_f, rwhh_b, rbias_b, rwsum_b,
      fca, fcb, fcbias)

    return out.reshape(WD, B, N_FEAT).transpose(1, 0, 2)


# speedup vs baseline: 1.0264x; 1.0258x over previous
import functools

import jax
import jax.numpy as jnp
from jax.experimental import pallas as pl
from jax.experimental.pallas import tpu as pltpu

N_FEAT = 64
WD = 128
B = 32
H = 64
CS = 8
S = 4


def _conv_kernel(xp_ref, w2_ref, b2_ref, w3_ref, b3_ref, o2_ref, o3_ref):
    for b in range(CS):
        xp = xp_ref[b]
        acc2 = jnp.broadcast_to(b2_ref[0], (WD, N_FEAT))
        for k in range(5):
            acc2 = acc2 + jnp.dot(xp[k + 1:k + 1 + WD, :], w2_ref[k],
                                  preferred_element_type=jnp.float32)
        o2_ref[b] = jnp.maximum(acc2, 0.0)
        acc3 = jnp.broadcast_to(b3_ref[0], (WD, N_FEAT))
        for k in range(7):
            acc3 = acc3 + jnp.dot(xp[k:k + WD, :], w3_ref[k],
                                  preferred_element_type=jnp.float32)
        o3_ref[b] = jnp.maximum(acc3, 0.0)


def _one_block(d, gwt, gas, gad, gb, cwt, cb, tri):
    xn = d.T
    h = jnp.dot(xn, gwt, preferred_element_type=jnp.float32)
    ht = h.T
    a_src = jnp.dot(gas, ht, preferred_element_type=jnp.float32)
    a_dst = jnp.dot(h, gad, preferred_element_type=jnp.float32)
    logits = a_dst + a_src
    logits = jnp.where(logits > 0, logits, 0.2 * logits)
    m = jnp.max(logits, axis=1, keepdims=True)
    e = jnp.exp(logits - m)
    att = e / jnp.sum(e, axis=1, keepdims=True)
    f = jnp.maximum(jnp.dot(att, h, preferred_element_type=jnp.float32)
                    + gb, 0.0)
    tin = f.T
    hh = jnp.dot(tin, cwt, preferred_element_type=jnp.float32)
    g = jnp.dot(tri, hh, preferred_element_type=jnp.float32)
    v = jnp.maximum(g + cb, 0.0)
    v3 = v.reshape(64, 2, 64)
    e0 = v3[:, 0, :].reshape(64, 64)
    e1 = v3[:, 1, :].reshape(64, 64)
    return jnp.concatenate([e0.T, e1.T], axis=0)


def _block_kernel(d_ref, gwt0, gas0, gad0, gb0, cwt0, cb0,
                  gwt1, gas1, gad1, gb1, cwt1, cb1, tri_ref, o_ref):
    tri = tri_ref[...]
    for b in range(S):
        d = d_ref[b]
        d = d + _one_block(d, gwt0[0], gas0[0], gad0[0], gb0[0],
                           cwt0[0], cb0[0], tri)
        d = d + _one_block(d, gwt1[0], gas1[0], gad1[0], gb1[0],
                           cwt1[0], cb1[0], tri)
        o_ref[b] = d


def _sigmoid(v):
    return jax.nn.sigmoid(v)


def _lstm_kernel(x_ref, wih_f, whh_f, bias_f, wih_b, bias_b,
                 rwhh_f, rbias_f, rwsum_f, rwhh_b, rbias_b, rwsum_b,
                 fca_ref, fcb_ref, fcbias_ref,
                 out_ref, xg_ref, hsf_ref, hsb_ref):
    for i in range(16):
        xg_ref[i * 256:(i + 1) * 256, :] = jnp.dot(
            x_ref[i * 256:(i + 1) * 256, :], wih_f[...],
            preferred_element_type=jnp.float32)

    zero = jnp.zeros((B, H), jnp.float32)

    def lstm_update(g, c):
        i = _sigmoid(g[:, 0:H])
        f = _sigmoid(g[:, H:2 * H])
        gg = jnp.tanh(g[:, 2 * H:3 * H])
        o = _sigmoid(g[:, 3 * H:4 * H])
        c2 = f * c + i * gg
        return o * jnp.tanh(c2), c2

    def enc_step(t, carry):
        h, c = carry
        g = (xg_ref[pl.ds(t * B, B), :]
             + jnp.dot(h, whh_f[...], preferred_element_type=jnp.float32)
             + bias_f[...])
        return lstm_update(g, c)

    h_f, _ = jax.lax.fori_loop(0, WD, enc_step, (zero, zero), unroll=4)

    gb = jnp.dot(x_ref[(WD - 1) * B:WD * B, :], wih_b[...],
                 preferred_element_type=jnp.float32) + bias_b[...]
    h_b, _ = lstm_update(gb, jnp.zeros((B, H), jnp.float32))

    ue = jnp.concatenate([h_f, h_b], axis=1)

    lane = jax.lax.broadcasted_iota(jnp.int32, (B, 2 * H), 1)

    def col(t):
        return jnp.sum(jnp.where(lane == t, ue, 0.0), axis=1, keepdims=True)

    def dec_step(k, carry):
        hf, cf, hb, cb = carry
        gf = (col(k) * rwsum_f[...]
              + jnp.dot(hf, rwhh_f[...], preferred_element_type=jnp.float32)
              + rbias_f[...])
        hf, cf = lstm_update(gf, cf)
        gbk = (col(WD - 1 - k) * rwsum_b[...]
               + jnp.dot(hb, rwhh_b[...], preferred_element_type=jnp.float32)
               + rbias_b[...])
        hb, cb = lstm_update(gbk, cb)
        hsf_ref[pl.ds(k * B, B), :] = hf
        hsb_ref[pl.ds((WD - 1 - k) * B, B), :] = hb
        return hf, cf, hb, cb

    jax.lax.fori_loop(0, WD, dec_step, (zero, zero, zero, zero), unroll=4)

    for i in range(8):
        sl = pl.ds(i * 512, 512)
        out_ref[sl, :] = (
            jnp.dot(hsf_ref[sl, :], fca_ref[...],
                    preferred_element_type=jnp.float32)
            + jnp.dot(hsb_ref[sl, :], fcb_ref[...],
                      preferred_element_type=jnp.float32)
            + fcbias_ref[...])


@functools.partial(jax.jit, static_argnames=())
def kernel(x, conv2_W, conv2_b, conv3_W, conv3_b, gat_W, gat_as, gat_ad,
           gat_b, gcn_W, gcn_b, lstm_Wih, lstm_Whh, lstm_bih, lstm_bhh,
           rec_Wih, rec_Whh, rec_bih, rec_bhh, fc_W, fc_b):
    f32 = jnp.float32

    xp = jnp.pad(x, ((0, 0), (3, 3), (0, 0)))
    w2t = jnp.transpose(conv2_W, (2, 1, 0))
    w3t = jnp.transpose(conv3_W, (2, 1, 0))
    b2 = conv2_b.reshape(1, 1, N_FEAT)
    b3 = conv3_b.reshape(1, 1, N_FEAT)
    x2, x3 = pl.pallas_call(
        _conv_kernel,
        grid=(B // CS,),
        in_specs=[
            pl.BlockSpec((CS, WD + 6, N_FEAT), lambda i: (i, 0, 0)),
            pl.BlockSpec((5, N_FEAT, N_FEAT), lambda i: (0, 0, 0)),
            pl.BlockSpec((1, 1, N_FEAT), lambda i: (0, 0, 0)),
            pl.BlockSpec((7, N_FEAT, N_FEAT), lambda i: (0, 0, 0)),
            pl.BlockSpec((1, 1, N_FEAT), lambda i: (0, 0, 0)),
        ],
        out_specs=[
            pl.BlockSpec((CS, WD, N_FEAT), lambda i: (i, 0, 0)),
            pl.BlockSpec((CS, WD, N_FEAT), lambda i: (i, 0, 0)),
        ],
        out_shape=[
            jax.ShapeDtypeStruct((B, WD, N_FEAT), f32),
            jax.ShapeDtypeStruct((B, WD, N_FEAT), f32),
        ],
        compiler_params=pltpu.CompilerParams(
            dimension_semantics=("parallel",)),
    )(xp, w2t, b2, w3t, b3)

    idx = jnp.arange(WD, dtype=f32)
    dinv = (idx + 1.0) ** -0.5
    tri = jnp.tril(jnp.ones((WD, WD), f32)) * (dinv[:, None] * dinv[None, :])

    data = jnp.stack([x, x2, x3]).reshape(3 * B, WD, N_FEAT)
    gwt = jnp.transpose(gat_W, (0, 2, 1))
    cwt = jnp.transpose(gcn_W, (0, 2, 1))

    nprog = 3 * B // S
    per_branch = B // S

    def wspec(shape):
        return pl.BlockSpec((1,) + shape, lambda i: (i // per_branch, 0, 0))

    def layer_args(l):
        return (
            gwt[l::2],
            gat_as[l::2].reshape(3, 1, WD),
            gat_ad[l::2].reshape(3, WD, 1),
            gat_b[l::2].reshape(3, 1, WD),
            cwt[l::2],
            gcn_b[l::2].reshape(3, 1, N_FEAT),
        )

    def layer_specs():
        return [
            wspec((WD, WD)),
            wspec((1, WD)),
            wspec((WD, 1)),
            wspec((1, WD)),
            wspec((N_FEAT, N_FEAT)),
            wspec((1, N_FEAT)),
        ]

    data = pl.pallas_call(
        _block_kernel,
        grid=(nprog,),
        in_specs=(
            [pl.BlockSpec((S, WD, N_FEAT), lambda i: (i, 0, 0))]
            + layer_specs() + layer_specs()
            + [pl.BlockSpec((WD, WD), lambda i: (0, 0))]
        ),
        out_specs=pl.BlockSpec((S, WD, N_FEAT), lambda i: (i, 0, 0)),
        out_shape=jax.ShapeDtypeStruct((3 * B, WD, N_FEAT), f32),
        compiler_params=pltpu.CompilerParams(
            dimension_semantics=("parallel",)),
    )(data, *layer_args(0), *layer_args(1), tri)

    xs = data.reshape(3, B, WD, N_FEAT).transpose(2, 1, 0, 3)
    xs = xs.reshape(WD * B, 3 * N_FEAT)

    wih_f = lstm_Wih[0].T
    whh_f = lstm_Whh[0].T
    bias_f = (lstm_bih[0] + lstm_bhh[0]).reshape(1, 4 * H)
    wih_b = lstm_Wih[1].T
    bias_b = (lstm_bih[1] + lstm_bhh[1]).reshape(1, 4 * H)

    rwhh_f = rec_Whh[0].T
    rbias_f = (rec_bih[0] + rec_bhh[0]).reshape(1, 4 * H)
    rwsum_f = jnp.sum(rec_Wih[0], axis=1).reshape(1, 4 * H)
    rwhh_b = rec_Whh[1].T
    rbias_b = (rec_bih[1] + rec_bhh[1]).reshape(1, 4 * H)
    rwsum_b = jnp.sum(rec_Wih[1], axis=1).reshape(1, 4 * H)

    fca = fc_W[:, :H].T
    fcb = fc_W[:, H:].T
    fcbias = fc_b.reshape(1, N_FEAT)

    out = pl.pallas_call(
        _lstm_kernel,
        out_shape=jax.ShapeDtypeStruct((WD * B, N_FEAT), f32),
        scratch_shapes=[
            pltpu.VMEM((WD * B, 4 * H), f32),
            pltpu.VMEM((WD * B, H), f32),
            pltpu.VMEM((WD * B, H), f32),
        ],
    )(xs, wih_f, whh_f, bias_f, wih_b, bias_b,
      rwhh_f, rbias_f, rwsum_f, rwhh_b, rbias_b, rwsum_b,
      fca, fcb, fcbias)

    return out.reshape(WD, B, N_FEAT).transpose(1, 0, 2)


# speedup vs baseline: 1.0308x; 1.0043x over previous
import functools

import jax
import jax.numpy as jnp
from jax.experimental import pallas as pl
from jax.experimental.pallas import tpu as pltpu

N_FEAT = 64
WD = 128
B = 32
H = 64
CS = 8
S = 4


def _conv_kernel(xp_ref, w2_ref, b2_ref, w3_ref, b3_ref, o2_ref, o3_ref):
    for b in range(CS):
        xp = xp_ref[b]
        acc2 = jnp.broadcast_to(b2_ref[0], (WD, N_FEAT))
        for k in range(5):
            acc2 = acc2 + jnp.dot(xp[k + 1:k + 1 + WD, :], w2_ref[k],
                                  preferred_element_type=jnp.float32)
        o2_ref[b] = jnp.maximum(acc2, 0.0)
        acc3 = jnp.broadcast_to(b3_ref[0], (WD, N_FEAT))
        for k in range(7):
            acc3 = acc3 + jnp.dot(xp[k:k + WD, :], w3_ref[k],
                                  preferred_element_type=jnp.float32)
        o3_ref[b] = jnp.maximum(acc3, 0.0)


def _one_block(d, gwt, gas, gad, gb, cwt, cb, tri):
    xn = d.T
    h = jnp.dot(xn, gwt, preferred_element_type=jnp.float32)
    ht = h.T
    a_src = jnp.dot(gas, ht, preferred_element_type=jnp.float32)
    a_dst = jnp.dot(h, gad, preferred_element_type=jnp.float32)
    logits = a_dst + a_src
    logits = jnp.where(logits > 0, logits, 0.2 * logits)
    m = jnp.max(logits, axis=1, keepdims=True)
    e = jnp.exp(logits - m)
    att = e / jnp.sum(e, axis=1, keepdims=True)
    f = jnp.maximum(jnp.dot(att, h, preferred_element_type=jnp.float32)
                    + gb, 0.0)
    tin = f.T
    hh = jnp.dot(tin, cwt, preferred_element_type=jnp.float32)
    g = jnp.dot(tri, hh, preferred_element_type=jnp.float32)
    v = jnp.maximum(g + cb, 0.0)
    v3 = v.reshape(64, 2, 64)
    e0 = v3[:, 0, :].reshape(64, 64)
    e1 = v3[:, 1, :].reshape(64, 64)
    return jnp.concatenate([e0.T, e1.T], axis=0)


def _block_kernel(d_ref, gwt0, gas0, gad0, gb0, cwt0, cb0,
                  gwt1, gas1, gad1, gb1, cwt1, cb1, tri_ref, o_ref):
    tri = tri_ref[...]
    for b in range(S):
        d = d_ref[b]
        d = d + _one_block(d, gwt0[0], gas0[0], gad0[0], gb0[0],
                           cwt0[0], cb0[0], tri)
        d = d + _one_block(d, gwt1[0], gas1[0], gad1[0], gb1[0],
                           cwt1[0], cb1[0], tri)
        o_ref[b] = d


def _sigmoid(v):
    return jax.nn.sigmoid(v)


def _lstm_kernel(x_ref, wih_f, whh_f, bias_f, wih_b, bias_b,
                 rwhh_f, rbias_f, rwsum_f, rwhh_b, rbias_b, rwsum_b,
                 fca_ref, fcb_ref, fcbias_ref,
                 out_ref, xg_ref, hsf_ref, hsb_ref):
    for i in range(16):
        xg_ref[i * 256:(i + 1) * 256, :] = jnp.dot(
            x_ref[i * 256:(i + 1) * 256, :], wih_f[...],
            preferred_element_type=jnp.float32)

    zero = jnp.zeros((B, H), jnp.float32)

    def lstm_update(g, c):
        i = _sigmoid(g[:, 0:H])
        f = _sigmoid(g[:, H:2 * H])
        gg = jnp.tanh(g[:, 2 * H:3 * H])
        o = _sigmoid(g[:, 3 * H:4 * H])
        c2 = f * c + i * gg
        return o * jnp.tanh(c2), c2

    def enc_step(t, carry):
        h, c = carry
        g = (xg_ref[pl.ds(t * B, B), :]
             + jnp.dot(h, whh_f[...], preferred_element_type=jnp.float32)
             + bias_f[...])
        return lstm_update(g, c)

    h_f, _ = jax.lax.fori_loop(0, WD, enc_step, (zero, zero), unroll=8)

    gb = jnp.dot(x_ref[(WD - 1) * B:WD * B, :], wih_b[...],
                 preferred_element_type=jnp.float32) + bias_b[...]
    h_b, _ = lstm_update(gb, jnp.zeros((B, H), jnp.float32))

    ue = jnp.concatenate([h_f, h_b], axis=1)

    lane = jax.lax.broadcasted_iota(jnp.int32, (B, 2 * H), 1)

    def col(t):
        return jnp.sum(jnp.where(lane == t, ue, 0.0), axis=1, keepdims=True)

    def dec_step(k, carry):
        hf, cf, hb, cb = carry
        gf = (col(k) * rwsum_f[...]
              + jnp.dot(hf, rwhh_f[...], preferred_element_type=jnp.float32)
              + rbias_f[...])
        hf, cf = lstm_update(gf, cf)
        gbk = (col(WD - 1 - k) * rwsum_b[...]
               + jnp.dot(hb, rwhh_b[...], preferred_element_type=jnp.float32)
               + rbias_b[...])
        hb, cb = lstm_update(gbk, cb)
        hsf_ref[pl.ds(k * B, B), :] = hf
        hsb_ref[pl.ds((WD - 1 - k) * B, B), :] = hb
        return hf, cf, hb, cb

    jax.lax.fori_loop(0, WD, dec_step, (zero, zero, zero, zero), unroll=8)

    for i in range(8):
        sl = pl.ds(i * 512, 512)
        out_ref[sl, :] = (
            jnp.dot(hsf_ref[sl, :], fca_ref[...],
                    preferred_element_type=jnp.float32)
            + jnp.dot(hsb_ref[sl, :], fcb_ref[...],
                      preferred_element_type=jnp.float32)
            + fcbias_ref[...])


@functools.partial(jax.jit, static_argnames=())
def kernel(x, conv2_W, conv2_b, conv3_W, conv3_b, gat_W, gat_as, gat_ad,
           gat_b, gcn_W, gcn_b, lstm_Wih, lstm_Whh, lstm_bih, lstm_bhh,
           rec_Wih, rec_Whh, rec_bih, rec_bhh, fc_W, fc_b):
    f32 = jnp.float32

    xp = jnp.pad(x, ((0, 0), (3, 3), (0, 0)))
    w2t = jnp.transpose(conv2_W, (2, 1, 0))
    w3t = jnp.transpose(conv3_W, (2, 1, 0))
    b2 = conv2_b.reshape(1, 1, N_FEAT)
    b3 = conv3_b.reshape(1, 1, N_FEAT)
    x2, x3 = pl.pallas_call(
        _conv_kernel,
        grid=(B // CS,),
        in_specs=[
            pl.BlockSpec((CS, WD + 6, N_FEAT), lambda i: (i, 0, 0)),
            pl.BlockSpec((5, N_FEAT, N_FEAT), lambda i: (0, 0, 0)),
            pl.BlockSpec((1, 1, N_FEAT), lambda i: (0, 0, 0)),
            pl.BlockSpec((7, N_FEAT, N_FEAT), lambda i: (0, 0, 0)),
            pl.BlockSpec((1, 1, N_FEAT), lambda i: (0, 0, 0)),
        ],
        out_specs=[
            pl.BlockSpec((CS, WD, N_FEAT), lambda i: (i, 0, 0)),
            pl.BlockSpec((CS, WD, N_FEAT), lambda i: (i, 0, 0)),
        ],
        out_shape=[
            jax.ShapeDtypeStruct((B, WD, N_FEAT), f32),
            jax.ShapeDtypeStruct((B, WD, N_FEAT), f32),
        ],
        compiler_params=pltpu.CompilerParams(
            dimension_semantics=("parallel",)),
    )(xp, w2t, b2, w3t, b3)

    idx = jnp.arange(WD, dtype=f32)
    dinv = (idx + 1.0) ** -0.5
    tri = jnp.tril(jnp.ones((WD, WD), f32)) * (dinv[:, None] * dinv[None, :])

    data = jnp.stack([x, x2, x3]).reshape(3 * B, WD, N_FEAT)
    gwt = jnp.transpose(gat_W, (0, 2, 1))
    cwt = jnp.transpose(gcn_W, (0, 2, 1))

    nprog = 3 * B // S
    per_branch = B // S

    def wspec(shape):
        return pl.BlockSpec((1,) + shape, lambda i: (i // per_branch, 0, 0))

    def layer_args(l):
        return (
            gwt[l::2],
            gat_as[l::2].reshape(3, 1, WD),
            gat_ad[l::2].reshape(3, WD, 1),
            gat_b[l::2].reshape(3, 1, WD),
            cwt[l::2],
            gcn_b[l::2].reshape(3, 1, N_FEAT),
        )

    def layer_specs():
        return [
            wspec((WD, WD)),
            wspec((1, WD)),
            wspec((WD, 1)),
            wspec((1, WD)),
            wspec((N_FEAT, N_FEAT)),
            wspec((1, N_FEAT)),
        ]

    data = pl.pallas_call(
        _block_kernel,
        grid=(nprog,),
        in_specs=(
            [pl.BlockSpec((S, WD, N_FEAT), lambda i: (i, 0, 0))]
            + layer_specs() + layer_specs()
            + [pl.BlockSpec((WD, WD), lambda i: (0, 0))]
        ),
        out_specs=pl.BlockSpec((S, WD, N_FEAT), lambda i: (i, 0, 0)),
        out_shape=jax.ShapeDtypeStruct((3 * B, WD, N_FEAT), f32),
        compiler_params=pltpu.CompilerParams(
            dimension_semantics=("parallel",)),
    )(data, *layer_args(0), *layer_args(1), tri)

    xs = data.reshape(3, B, WD, N_FEAT).transpose(2, 1, 0, 3)
    xs = xs.reshape(WD * B, 3 * N_FEAT)

    wih_f = lstm_Wih[0].T
    whh_f = lstm_Whh[0].T
    bias_f = (lstm_bih[0] + lstm_bhh[0]).reshape(1, 4 * H)
    wih_b = lstm_Wih[1].T
    bias_b = (lstm_bih[1] + lstm_bhh[1]).reshape(1, 4 * H)

    rwhh_f = rec_Whh[0].T
    rbias_f = (rec_bih[0] + rec_bhh[0]).reshape(1, 4 * H)
    rwsum_f = jnp.sum(rec_Wih[0], axis=1).reshape(1, 4 * H)
    rwhh_b = rec_Whh[1].T
    rbias_b = (rec_bih[1] + rec_bhh[1]).reshape(1, 4 * H)
    rwsum_b = jnp.sum(rec_Wih[1], axis=1).reshape(1, 4 * H)

    fca = fc_W[:, :H].T
    fcb = fc_W[:, H:].T
    fcbias = fc_b.reshape(1, N_FEAT)

    out = pl.pallas_call(
        _lstm_kernel,
        out_shape=jax.ShapeDtypeStruct((WD * B, N_FEAT), f32),
        scratch_shapes=[
            pltpu.VMEM((WD * B, 4 * H), f32),
            pltpu.VMEM((WD * B, H), f32),
            pltpu.VMEM((WD * B, H), f32),
        ],
    )(xs, wih_f, whh_f, bias_f, wih_b, bias_b,
      rwhh_f, rbias_f, rwsum_f, rwhh_b, rbias_b, rwsum_b,
      fca, fcb, fcbias)

    return out.reshape(WD, B, N_FEAT).transpose(1, 0, 2)
